# baseline (speedup 1.0000x reference)
import functools

import jax
import jax.numpy as jnp
from jax import lax
from jax.experimental import pallas as pl
from jax.experimental.pallas import tpu as pltpu
from jax.experimental.pallas import tpu_sc as plsc

_CHUNK = 128


@functools.partial(jax.jit, static_argnames=("b_per_w", "n_chunks", "dim"))
def _sc_lookup(iu, iv, U, V, *, b_per_w, n_chunks, dim):
    B = iu.shape[0] * iu.shape[1] * iu.shape[2] // 1
    NW = iu.shape[0]
    info = plsc.get_sparse_core_info()
    NC = info.num_cores

    mesh = plsc.VectorSubcoreMesh(core_axis_name="c", subcore_axis_name="s")

    @functools.partial(
        pl.kernel,
        mesh=mesh,
        out_type=(
            jax.ShapeDtypeStruct((NW * b_per_w, dim), jnp.float32),
            jax.ShapeDtypeStruct((NW * b_per_w, dim), jnp.float32),
        ),
        scratch_types=[
            pltpu.VMEM((n_chunks, _CHUNK), jnp.int32),
            pltpu.VMEM((n_chunks, _CHUNK), jnp.int32),
            pltpu.VMEM((b_per_w, dim), jnp.float32),
            pltpu.VMEM((b_per_w, dim), jnp.float32),
            pltpu.SemaphoreType.DMA,
        ],
        compiler_params=pltpu.CompilerParams(use_tc_tiling_on_sc=False),
    )
    def k(iu_hbm, iv_hbm, U_hbm, V_hbm, ou_hbm, ov_hbm, iu_v, iv_v, ru_v, rv_v, sem):
        wid = lax.axis_index("s") * NC + lax.axis_index("c")
        base = wid * b_per_w
        pltpu.sync_copy(iu_hbm.at[wid], iu_v)
        pltpu.sync_copy(iv_hbm.at[wid], iv_v)
        copies = []
        for c in range(n_chunks):
            copies.append(
                pltpu.async_copy(
                    U_hbm.at[iu_v.at[c]], ru_v.at[pl.ds(c * _CHUNK, _CHUNK)], sem
                )
            )
            copies.append(
                pltpu.async_copy(
                    V_hbm.at[iv_v.at[c]], rv_v.at[pl.ds(c * _CHUNK, _CHUNK)], sem
                )
            )
        for cp in copies:
            cp.wait()
        pltpu.sync_copy(ru_v, ou_hbm.at[pl.ds(base, b_per_w)])
        pltpu.sync_copy(rv_v, ov_hbm.at[pl.ds(base, b_per_w)])

    return k(iu, iv, U, V)


def kernel(ij, U, V):
    B = ij.shape[0]
    dim = U.shape[1]
    info = plsc.get_sparse_core_info()
    NW = info.num_cores * info.num_subcores
    b_per_w = B // NW
    n_chunks = b_per_w // _CHUNK
    idx = ij.astype(jnp.int32)
    iu = idx[:, 0].reshape(NW, n_chunks, _CHUNK)
    iv = idx[:, 1].reshape(NW, n_chunks, _CHUNK)
    return _sc_lookup(iu, iv, U, V, b_per_w=b_per_w, n_chunks=n_chunks, dim=dim)
